# baseline (speedup 1.0000x reference)
import functools

import jax
import jax.numpy as jnp
from jax import lax
from jax.experimental import pallas as pl
from jax.experimental.pallas import tpu as pltpu
from jax.experimental.pallas import tpu_sc as plsc

BATCH = 16384
D = 64
NC = 2
NS = 16
NW = NC * NS
BPW = BATCH // NW
CHUNK = 128
NCHUNK = BPW // CHUNK
GROUPS = BPW // 16


def _body(user_hbm, ii_hbm, ij_hbm, eu_hbm, ei_hbm, out_i_hbm, out_j_hbm,
          uidx_v, iidx_v, jidx_v, urows_v, virows_v, vjrows_v,
          pmat_v, outi_v, outj_v, sems):
    wid = lax.axis_index("s") * NC + lax.axis_index("c")
    base = wid * BPW

    for c in range(NCHUNK):
        off = base + c * CHUNK
        pltpu.sync_copy(user_hbm.at[pl.ds(off, CHUNK)], uidx_v.at[c])
        pltpu.sync_copy(ii_hbm.at[pl.ds(off, CHUNK)], iidx_v.at[c])
        pltpu.sync_copy(ij_hbm.at[pl.ds(off, CHUNK)], jidx_v.at[c])

    copies = []
    for c in range(NCHUNK):
        rows = pl.ds(c * CHUNK, CHUNK)
        copies.append(pltpu.async_copy(
            eu_hbm.at[uidx_v.at[c]], urows_v.at[rows], sems.at[0]))
        copies.append(pltpu.async_copy(
            ei_hbm.at[iidx_v.at[c]], virows_v.at[rows], sems.at[1]))
        copies.append(pltpu.async_copy(
            ei_hbm.at[jidx_v.at[c]], vjrows_v.at[rows], sems.at[2]))
    for cp in copies:
        cp.wait()

    lanes = lax.iota(jnp.int32, 16)

    def group(g, carry):
        gb = g * 16
        for s in range(16):
            row = gb + s
            u0 = urows_v[row, pl.ds(0, 16)]
            u1 = urows_v[row, pl.ds(16, 16)]
            u2 = urows_v[row, pl.ds(32, 16)]
            u3 = urows_v[row, pl.ds(48, 16)]
            pi = (u0 * virows_v[row, pl.ds(0, 16)]
                  + u1 * virows_v[row, pl.ds(16, 16)]
                  + u2 * virows_v[row, pl.ds(32, 16)]
                  + u3 * virows_v[row, pl.ds(48, 16)])
            pj = (u0 * vjrows_v[row, pl.ds(0, 16)]
                  + u1 * vjrows_v[row, pl.ds(16, 16)]
                  + u2 * vjrows_v[row, pl.ds(32, 16)]
                  + u3 * vjrows_v[row, pl.ds(48, 16)])
            col = lanes * 16 + s
            plsc.store_scatter(pmat_v, [col], pi)
            plsc.store_scatter(pmat_v, [col + 256], pj)
        acc_i = pmat_v[pl.ds(0, 16)]
        acc_j = pmat_v[pl.ds(256, 16)]
        for l in range(1, 16):
            acc_i = acc_i + pmat_v[pl.ds(l * 16, 16)]
            acc_j = acc_j + pmat_v[pl.ds(256 + l * 16, 16)]
        outi_v[pl.ds(gb, 16)] = acc_i
        outj_v[pl.ds(gb, 16)] = acc_j
        return carry

    lax.fori_loop(0, GROUPS, group, 0)

    pltpu.sync_copy(outi_v, out_i_hbm.at[pl.ds(base, BPW)])
    pltpu.sync_copy(outj_v, out_j_hbm.at[pl.ds(base, BPW)])


@jax.jit
def _mf_forward(user, item_i, item_j, embed_user, embed_item):
    mesh = plsc.VectorSubcoreMesh(core_axis_name="c", subcore_axis_name="s")
    f = functools.partial(
        pl.kernel,
        out_type=(jax.ShapeDtypeStruct((BATCH,), jnp.float32),
                  jax.ShapeDtypeStruct((BATCH,), jnp.float32)),
        mesh=mesh,
        scratch_types=[
            pltpu.VMEM((NCHUNK, CHUNK), jnp.int32),
            pltpu.VMEM((NCHUNK, CHUNK), jnp.int32),
            pltpu.VMEM((NCHUNK, CHUNK), jnp.int32),
            pltpu.VMEM((BPW, D), jnp.float32),
            pltpu.VMEM((BPW, D), jnp.float32),
            pltpu.VMEM((BPW, D), jnp.float32),
            pltpu.VMEM((512,), jnp.float32),
            pltpu.VMEM((BPW,), jnp.float32),
            pltpu.VMEM((BPW,), jnp.float32),
            pltpu.SemaphoreType.DMA((3,)),
        ],
        compiler_params=pltpu.CompilerParams(
            needs_layout_passes=False, use_tc_tiling_on_sc=False),
    )(_body)
    return f(user, item_i, item_j, embed_user, embed_item)


def kernel(user, item_i, item_j, embed_user, embed_item):
    return _mf_forward(user, item_i, item_j, embed_user, embed_item)
